# Initial kernel scaffold; baseline (speedup 1.0000x reference)
import jax
import jax.numpy as jnp
from jax.experimental import pallas as pl


def kernel(x, edge_attr, params, edge_index):
    raise NotImplementedError("write your pallas kernel here")


# speedup vs baseline: 3.1116x; 3.1116x over previous
import functools

import jax
import jax.numpy as jnp
from jax import lax
from jax.experimental import pallas as pl
from jax.experimental.pallas import tpu as pltpu
from jax.experimental.pallas import tpu_sc as plsc

N = 100000
E = 1600000
D = 16

CH = 128
SUP = 8
NSUP = 49
NW = 32
EPW = CH * SUP * NSUP
E_PAD = EPW * NW
G2 = E_PAD // CH

BE = 2048
GE = E_PAD // BE
BN = 4000
GN = N // BN

SP_ROWS = 100224
ZR = SP_ROWS // 16
NPR = N // 16

_mesh = plsc.VectorSubcoreMesh(core_axis_name="c", subcore_axis_name="s")
_sc_params = pltpu.CompilerParams(use_tc_tiling_on_sc=False)


def _gather_body(table, srcr, dstr, xs3, xd3, sidx, didx, gs, gd, sem):
    c = lax.axis_index("c")
    s = lax.axis_index("s")
    wid = s * 2 + c
    r0 = wid * (SUP * NSUP)

    def step(i, carry):
        row = r0 + i * SUP
        pltpu.sync_copy(srcr.at[pl.ds(row, SUP)], sidx)
        pltpu.sync_copy(dstr.at[pl.ds(row, SUP)], didx)
        cps = []
        for j in range(SUP):
            cps.append(pltpu.async_copy(table.at[sidx.at[j]], gs.at[j], sem))
            cps.append(pltpu.async_copy(table.at[didx.at[j]], gd.at[j], sem))
        for cp in cps:
            cp.wait()
        pltpu.sync_copy(gs, xs3.at[pl.ds(row, SUP)])
        pltpu.sync_copy(gd, xd3.at[pl.ds(row, SUP)])
        return carry

    lax.fori_loop(0, NSUP, step, 0)


_gather_call = pl.kernel(
    _gather_body,
    out_type=(
        jax.ShapeDtypeStruct((G2, CH, D), jnp.float32),
        jax.ShapeDtypeStruct((G2, CH, D), jnp.float32),
    ),
    mesh=_mesh,
    scratch_types=[
        pltpu.VMEM((SUP, CH), jnp.int32),
        pltpu.VMEM((SUP, CH), jnp.int32),
        pltpu.VMEM((SUP, CH, D), jnp.float32),
        pltpu.VMEM((SUP, CH, D), jnp.float32),
        pltpu.SemaphoreType.DMA,
    ],
    compiler_params=_sc_params,
)


def _scatter_body(mr, dstr, zer, agg2, idxb, vbuf, shared, sem):
    c = lax.axis_index("c")
    s = lax.axis_index("s")
    wid = s * 2 + c
    r0 = wid * (SUP * NSUP)

    pltpu.sync_copy(zer.at[pl.ds(s * ZR, ZR)], shared.at[pl.ds(s * ZR, ZR)])
    plsc.subcore_barrier()

    def step(i, carry):
        row = r0 + i * SUP
        pltpu.sync_copy(dstr.at[pl.ds(row, SUP)], idxb)
        pltpu.sync_copy(mr.at[pl.ds(row, SUP)], vbuf)
        for j in range(SUP):
            pltpu.sync_copy(vbuf.at[j], shared.at[idxb.at[j]], add=True)
        return carry

    lax.fori_loop(0, NSUP, step, 0)
    plsc.subcore_barrier()
    pltpu.sync_copy(shared.at[pl.ds(s * NPR, NPR)], agg2.at[c, pl.ds(s * NPR, NPR)])


_scatter_call = pl.kernel(
    _scatter_body,
    out_type=jax.ShapeDtypeStruct((2, N, 8), jnp.float32),
    mesh=_mesh,
    scratch_types=[
        pltpu.VMEM((SUP, CH), jnp.int32),
        pltpu.VMEM((SUP, CH, 8), jnp.float32),
        pltpu.VMEM_SHARED((SP_ROWS, 8), jnp.float32),
        pltpu.SemaphoreType.DMA,
    ],
    compiler_params=_sc_params,
)


def _edge_body(xs, xd, ea,
               ec_a, ec_b, ec_c, ec_b1, ec_w2, ec_b2,
               n1, n2, e1, e2,
               ph_a, ph_b, ph_c, ph_b1, ph_w2, ph_b2,
               w_o, mask_o, m_o):
    f32 = jnp.float32
    xs_ = xs[...]
    xd_ = xd[...]
    ea_ = ea[...]
    dot = functools.partial(jnp.dot, preferred_element_type=f32)
    h1 = jnp.maximum(dot(xs_, ec_a[...]) + dot(xd_, ec_b[...]) + dot(ea_, ec_c[...]) + ec_b1[...], 0.0)
    w = jax.nn.sigmoid(dot(h1, ec_w2[...]) + ec_b2[...])
    maskf = (w > 0.5).astype(f32)
    hs = jnp.maximum(dot(jnp.maximum(dot(xs_, n1[...]), 0.0), n2[...]), 0.0)
    hd = jnp.maximum(dot(jnp.maximum(dot(xd_, n1[...]), 0.0), n2[...]), 0.0)
    ehc = jnp.maximum(dot(jnp.maximum(dot(ea_, e1[...]), 0.0), e2[...]), 0.0)
    m1 = jnp.maximum(dot(hs, ph_a[...]) + dot(hd, ph_b[...]) + dot(ehc, ph_c[...]) + ph_b1[...], 0.0)
    m = (dot(m1, ph_w2[...]) + ph_b2[...]) * maskf
    w_o[...] = w
    mask_o[...] = maskf
    m_o[...] = m


def _full_spec(a):
    nd = a.ndim
    return pl.BlockSpec(a.shape, lambda i, _nd=nd: (0,) * _nd)


def _node_body(x16, agg2,
               n1, n2,
               hh_a, hh_b, hh_b1, hh_w2, hh_b2,
               pb1, pbb1, pb2, pbb2, pb3, pbb3,
               pc1, pcb1, pc2, pcb2, pc3, pcb3,
               beta_o, h_o):
    f32 = jnp.float32
    dot = functools.partial(jnp.dot, preferred_element_type=f32)
    x_ = x16[...]
    agg = agg2[0] + agg2[1]
    h_hc = jnp.maximum(dot(jnp.maximum(dot(x_, n1[...]), 0.0), n2[...]), 0.0)
    hn1 = jnp.maximum(dot(h_hc, hh_a[...]) + dot(agg, hh_b[...]) + hh_b1[...], 0.0)
    h_new = dot(hn1, hh_w2[...]) + hh_b2[...]
    b = jnp.maximum(dot(h_new, pb1[...]) + pbb1[...], 0.0)
    b = jnp.maximum(dot(b, pb2[...]) + pbb2[...], 0.0)
    beta_o[...] = jax.nn.sigmoid(dot(b, pb3[...]) + pbb3[...]) + 1e-8
    cc = jnp.maximum(dot(h_new, pc1[...]) + pcb1[...], 0.0)
    cc = jnp.maximum(dot(cc, pc2[...]) + pcb2[...], 0.0)
    h_o[...] = dot(cc, pc3[...]) + pcb3[...]


def kernel(x, edge_attr, params, edge_index):
    p = params
    f32 = jnp.float32
    src = edge_index[0]
    dst = edge_index[1]

    x16 = jnp.pad(x, ((0, 0), (0, D - x.shape[1])))
    pad = E_PAD - E
    src_p = jnp.concatenate([src, jnp.zeros((pad,), jnp.int32)]).reshape(G2, CH)
    dst_g = jnp.concatenate([dst, jnp.zeros((pad,), jnp.int32)]).reshape(G2, CH)
    dst_s = jnp.concatenate([dst, jnp.full((pad,), N, jnp.int32)]).reshape(G2, CH)
    ea_p = jnp.pad(edge_attr, ((0, pad), (0, 0)))

    xs3, xd3 = _gather_call(x16, src_p, dst_g)
    xs = xs3.reshape(E_PAD, D)
    xd = xd3.reshape(E_PAD, D)

    z16 = jnp.zeros((D, 40), f32)
    ec_a = z16.at[:14].set(p['ec_w1'][:14])
    ec_b = z16.at[:14].set(p['ec_w1'][14:28])
    ec_c = p['ec_w1'][28:32]
    n1 = z16.at[:14].set(p['nenc_w1'])
    ph_a = p['phie_w1'][:5]
    ph_b = p['phie_w1'][5:10]
    ph_c = p['phie_w1'][10:14]
    hh_a = p['phih_w1'][:5]
    hh_b = jnp.zeros((8, 40), f32).at[:4].set(p['phih_w1'][5:9])
    ph_w2_8 = jnp.zeros((40, 8), f32).at[:, :4].set(p['phie_w2'])
    ph_b2_8 = jnp.zeros((1, 8), f32).at[:, :4].set(p['phie_b2'].reshape(1, -1))
    r2 = lambda a: a.reshape(1, -1)

    ew_ins = [ec_a, ec_b, ec_c, r2(p['ec_b1']), p['ec_w2'], r2(p['ec_b2']),
              n1, p['nenc_w2'], p['eenc_w1'], p['eenc_w2'],
              ph_a, ph_b, ph_c, r2(p['phie_b1']), ph_w2_8, ph_b2_8]
    w_full, maskf, m = pl.pallas_call(
        _edge_body,
        grid=(GE,),
        in_specs=[pl.BlockSpec((BE, D), lambda i: (i, 0)),
                  pl.BlockSpec((BE, D), lambda i: (i, 0)),
                  pl.BlockSpec((BE, 4), lambda i: (i, 0))]
                 + [_full_spec(a) for a in ew_ins],
        out_specs=[pl.BlockSpec((BE, 1), lambda i: (i, 0)),
                   pl.BlockSpec((BE, 1), lambda i: (i, 0)),
                   pl.BlockSpec((BE, 8), lambda i: (i, 0))],
        out_shape=[jax.ShapeDtypeStruct((E_PAD, 1), f32),
                   jax.ShapeDtypeStruct((E_PAD, 1), f32),
                   jax.ShapeDtypeStruct((E_PAD, 8), f32)],
    )(xs, xd, ea_p, *ew_ins)

    zer = jnp.zeros((SP_ROWS, 8), f32)
    agg2 = _scatter_call(m.reshape(G2, CH, 8), dst_s, zer)

    nw_ins = [n1, p['nenc_w2'],
              hh_a, hh_b, r2(p['phih_b1']), p['phih_w2'], r2(p['phih_b2']),
              p['pb_w1'], r2(p['pb_b1']), p['pb_w2'], r2(p['pb_b2']), p['pb_w3'], r2(p['pb_b3']),
              p['pc_w1'], r2(p['pc_b1']), p['pc_w2'], r2(p['pc_b2']), p['pc_w3'], r2(p['pc_b3'])]
    beta, H = pl.pallas_call(
        _node_body,
        grid=(GN,),
        in_specs=[pl.BlockSpec((BN, D), lambda i: (i, 0)),
                  pl.BlockSpec((2, BN, 8), lambda i: (0, i, 0))]
                 + [_full_spec(a) for a in nw_ins],
        out_specs=[pl.BlockSpec((BN, 1), lambda i: (i, 0)),
                   pl.BlockSpec((BN, 2), lambda i: (i, 0))],
        out_shape=[jax.ShapeDtypeStruct((N, 1), f32),
                   jax.ShapeDtypeStruct((N, 2), f32)],
    )(x16, agg2, *nw_ins)

    edge_weights = w_full[:E]
    edge_mask = maskf[:E, 0].astype(bool)
    hit_mask = jnp.ones((N,), bool)
    return (edge_weights, H, beta, hit_mask, edge_mask)


# speedup vs baseline: 11.1684x; 3.5893x over previous
import functools

import jax
import jax.numpy as jnp
from jax import lax
from jax.experimental import pallas as pl
from jax.experimental.pallas import tpu as pltpu
from jax.experimental.pallas import tpu_sc as plsc

N = 100000
E = 1600000
D = 16

CH = 128
SUP = 8
NSUP = 49
NW = 32
EPW = CH * SUP * NSUP
E_PAD = EPW * NW
G2 = E_PAD // CH

BE = 4096
GE = E_PAD // BE
BN = 4000
GN = N // BN

SP_ROWS = 100224
ZR = SP_ROWS // 16
NPR = N // 16

_mesh = plsc.VectorSubcoreMesh(core_axis_name="c", subcore_axis_name="s")
_sc_params = pltpu.CompilerParams(use_tc_tiling_on_sc=False, needs_layout_passes=False)


def _gather_body(table, srcr, dstr, ea4, xs3, xd3, eapk, sidx, didx, gs, gd, eabuf, pk1, sem):
    c = lax.axis_index("c")
    s = lax.axis_index("s")
    wid = s * 2 + c
    r0 = wid * (SUP * NSUP)

    def step(i, carry):
        row = r0 + i * SUP
        base = row * CH
        pltpu.sync_copy(srcr.at[pl.ds(row, SUP)], sidx)
        pltpu.sync_copy(dstr.at[pl.ds(row, SUP)], didx)
        pltpu.sync_copy(ea4.at[:, pl.ds(base, SUP * CH)], eabuf)
        cps = []
        for j in range(SUP):
            cps.append(pltpu.async_copy(table.at[sidx.at[j]], gs.at[j], sem))
            cps.append(pltpu.async_copy(table.at[didx.at[j]], gd.at[j], sem))
        lanes = lax.iota(jnp.int32, 16)
        for f in range(4):
            for v in range(SUP * CH // 16):
                vec = eabuf[f, pl.ds(16 * v, 16)]
                idx = lanes * 16 + (256 * v + f)
                plsc.store_scatter(pk1, [idx], vec)
        for cp in cps:
            cp.wait()
        pltpu.sync_copy(gs, xs3.at[pl.ds(row, SUP)])
        pltpu.sync_copy(gd, xd3.at[pl.ds(row, SUP)])
        pltpu.sync_copy(pk1, eapk.at[pl.ds(base * 16, SUP * CH * 16)])
        return carry

    lax.fori_loop(0, NSUP, step, 0)


_gather_call = pl.kernel(
    _gather_body,
    out_type=(
        jax.ShapeDtypeStruct((G2, CH, D), jnp.float32),
        jax.ShapeDtypeStruct((G2, CH, D), jnp.float32),
        jax.ShapeDtypeStruct((E_PAD * 16,), jnp.float32),
    ),
    mesh=_mesh,
    scratch_types=[
        pltpu.VMEM((SUP, CH), jnp.int32),
        pltpu.VMEM((SUP, CH), jnp.int32),
        pltpu.VMEM((SUP, CH, D), jnp.float32),
        pltpu.VMEM((SUP, CH, D), jnp.float32),
        pltpu.VMEM((4, SUP * CH), jnp.float32),
        pltpu.VMEM((SUP * CH * 16,), jnp.float32),
        pltpu.SemaphoreType.DMA,
    ],
    compiler_params=_sc_params,
)


def _scatter_body(mr, dstr, zer, agg2, idxb, vbuf, shared, sem):
    c = lax.axis_index("c")
    s = lax.axis_index("s")
    wid = s * 2 + c
    r0 = wid * (SUP * NSUP)

    pltpu.sync_copy(zer.at[pl.ds(s * ZR, ZR)], shared.at[pl.ds(s * ZR, ZR)])
    plsc.subcore_barrier()

    def step(i, carry):
        row = r0 + i * SUP
        pltpu.sync_copy(dstr.at[pl.ds(row, SUP)], idxb)
        pltpu.sync_copy(mr.at[pl.ds(row, SUP)], vbuf)
        for j in range(SUP):
            pltpu.sync_copy(vbuf.at[j], shared.at[idxb.at[j]], add=True)
        return carry

    lax.fori_loop(0, NSUP, step, 0)
    plsc.subcore_barrier()
    pltpu.sync_copy(shared.at[pl.ds(s * NPR, NPR)], agg2.at[c, pl.ds(s * NPR, NPR)])


_scatter_call = pl.kernel(
    _scatter_body,
    out_type=jax.ShapeDtypeStruct((2, N, 16), jnp.float32),
    mesh=_mesh,
    scratch_types=[
        pltpu.VMEM((SUP, CH), jnp.int32),
        pltpu.VMEM((SUP, CH, 16), jnp.float32),
        pltpu.VMEM_SHARED((SP_ROWS, 16), jnp.float32),
        pltpu.SemaphoreType.DMA,
    ],
    compiler_params=_sc_params,
)


def _edge_body(xs, xd, ea,
               ec_a, ec_b, ec_c, ec_b1, ec_w2, ec_b2, ec_w2d, ec_b2d,
               n1, n2, e1, e2,
               ph_a, ph_b, ph_c, ph_b1, ph_w2, ph_b2,
               w_o, mask_o, m_o):
    f32 = jnp.float32
    xs_ = xs[...]
    xd_ = xd[...]
    ea_ = ea[...]
    dot = functools.partial(jnp.dot, preferred_element_type=f32)
    h1 = jnp.maximum(dot(xs_, ec_a[...]) + dot(xd_, ec_b[...]) + dot(ea_, ec_c[...]) + ec_b1[...], 0.0)
    w = jax.nn.sigmoid(dot(h1, ec_w2[...]) + ec_b2[...])
    maskf = (w > 0.5).astype(f32)
    w128 = jax.nn.sigmoid(dot(h1, ec_w2d[...]) + ec_b2d[...])
    maskf128 = (w128 > 0.5).astype(f32)
    hs = jnp.maximum(dot(jnp.maximum(dot(xs_, n1[...]), 0.0), n2[...]), 0.0)
    hd = jnp.maximum(dot(jnp.maximum(dot(xd_, n1[...]), 0.0), n2[...]), 0.0)
    ehc = jnp.maximum(dot(jnp.maximum(dot(ea_, e1[...]), 0.0), e2[...]), 0.0)
    m1 = jnp.maximum(dot(hs, ph_a[...]) + dot(hd, ph_b[...]) + dot(ehc, ph_c[...]) + ph_b1[...], 0.0)
    m = (dot(m1, ph_w2[...]) + ph_b2[...]) * maskf128
    w_o[...] = w
    mask_o[...] = maskf
    m_o[...] = m


def _full_spec(a):
    nd = a.ndim
    return pl.BlockSpec(a.shape, lambda i, _nd=nd: (0,) * _nd)


def _node_body(x16, agg2,
               n1, n2,
               hh_a, hh_b, hh_b1, hh_w2, hh_b2,
               pb1, pbb1, pb2, pbb2, pb3, pbb3,
               pc1, pcb1, pc2, pcb2, pc3, pcb3,
               beta_o, h_o):
    f32 = jnp.float32
    dot = functools.partial(jnp.dot, preferred_element_type=f32)
    x_ = x16[...]
    agg = agg2[0] + agg2[1]
    h_hc = jnp.maximum(dot(jnp.maximum(dot(x_, n1[...]), 0.0), n2[...]), 0.0)
    hn1 = jnp.maximum(dot(h_hc, hh_a[...]) + dot(agg, hh_b[...]) + hh_b1[...], 0.0)
    h_new = dot(hn1, hh_w2[...]) + hh_b2[...]
    b = jnp.maximum(dot(h_new, pb1[...]) + pbb1[...], 0.0)
    b = jnp.maximum(dot(b, pb2[...]) + pbb2[...], 0.0)
    beta_o[...] = jax.nn.sigmoid(dot(b, pb3[...]) + pbb3[...]) + 1e-8
    cc = jnp.maximum(dot(h_new, pc1[...]) + pcb1[...], 0.0)
    cc = jnp.maximum(dot(cc, pc2[...]) + pcb2[...], 0.0)
    h_o[...] = dot(cc, pc3[...]) + pcb3[...]


def kernel(x, edge_attr, params, edge_index):
    p = params
    f32 = jnp.float32
    src = edge_index[0]
    dst = edge_index[1]

    x16 = jnp.pad(x, ((0, 0), (0, D - x.shape[1])))
    pad = E_PAD - E
    src_p = jnp.concatenate([src, jnp.zeros((pad,), jnp.int32)]).reshape(G2, CH)
    dst_g = jnp.concatenate([dst, jnp.zeros((pad,), jnp.int32)]).reshape(G2, CH)
    dst_s = jnp.concatenate([dst, jnp.full((pad,), N, jnp.int32)]).reshape(G2, CH)
    ea_t = jnp.pad(edge_attr.T, ((0, 0), (0, pad)))

    xs3, xd3, eapk = _gather_call(x16, src_p, dst_g, ea_t)
    xs = xs3.reshape(E_PAD * D // 128, 128)
    xd = xd3.reshape(E_PAD * D // 128, 128)
    ea_pk = eapk.reshape(E_PAD // 8, 128)

    z16 = jnp.zeros((D, 40), f32)
    ec_a = z16.at[:14].set(p['ec_w1'][:14])
    ec_b = z16.at[:14].set(p['ec_w1'][14:28])
    ec_c = jnp.zeros((16, 40), f32).at[:4].set(p['ec_w1'][28:32])
    n1 = z16.at[:14].set(p['nenc_w1'])
    ph_a = p['phie_w1'][:5]
    ph_b = p['phie_w1'][5:10]
    ph_c = p['phie_w1'][10:14]
    hh_a = p['phih_w1'][:5]
    hh_b = jnp.zeros((16, 40), f32).at[:4].set(p['phih_w1'][5:9])
    ph_w2_8 = jnp.zeros((40, 16), f32).at[:, :4].set(p['phie_w2'])
    ph_b2_8 = jnp.zeros((1, 16), f32).at[:, :4].set(p['phie_b2'].reshape(1, -1))
    r2 = lambda a: a.reshape(1, -1)
    i8 = jnp.eye(8, dtype=f32)
    bd = lambda a: jnp.kron(i8, a)
    t8 = lambda a: jnp.tile(a.reshape(1, -1), (1, 8))

    ew_ins = [bd(ec_a), bd(ec_b), bd(ec_c), t8(p['ec_b1']), bd(p['ec_w2']), t8(p['ec_b2']),
              bd(jnp.tile(p['ec_w2'], (1, 16))), jnp.tile(p['ec_b2'].reshape(1, 1), (1, 128)),
              bd(n1), bd(p['nenc_w2']), bd(jnp.zeros((16, 40), f32).at[:4].set(p['eenc_w1'])), bd(p['eenc_w2']),
              bd(ph_a), bd(ph_b), bd(ph_c), t8(p['phie_b1']), bd(ph_w2_8), t8(ph_b2_8)]
    w_full, maskf, m = pl.pallas_call(
        _edge_body,
        grid=(GE,),
        in_specs=[pl.BlockSpec((BE * D // 128, 128), lambda i: (i, 0)),
                  pl.BlockSpec((BE * D // 128, 128), lambda i: (i, 0)),
                  pl.BlockSpec((BE // 8, 128), lambda i: (i, 0))]
                 + [_full_spec(a) for a in ew_ins],
        out_specs=[pl.BlockSpec((BE // 8, 8), lambda i: (i, 0)),
                   pl.BlockSpec((BE // 8, 8), lambda i: (i, 0)),
                   pl.BlockSpec((BE // 8, 128), lambda i: (i, 0))],
        out_shape=[jax.ShapeDtypeStruct((E_PAD // 8, 8), f32),
                   jax.ShapeDtypeStruct((E_PAD // 8, 8), f32),
                   jax.ShapeDtypeStruct((E_PAD // 8, 128), f32)],
    )(xs, xd, ea_pk, *ew_ins)

    zer = jnp.zeros((SP_ROWS, 16), f32)
    agg2 = _scatter_call(m.reshape(G2, CH, 16), dst_s, zer)

    nw_ins = [n1, p['nenc_w2'],
              hh_a, hh_b, r2(p['phih_b1']), p['phih_w2'], r2(p['phih_b2']),
              p['pb_w1'], r2(p['pb_b1']), p['pb_w2'], r2(p['pb_b2']), p['pb_w3'], r2(p['pb_b3']),
              p['pc_w1'], r2(p['pc_b1']), p['pc_w2'], r2(p['pc_b2']), p['pc_w3'], r2(p['pc_b3'])]
    beta, H = pl.pallas_call(
        _node_body,
        grid=(GN,),
        in_specs=[pl.BlockSpec((BN, D), lambda i: (i, 0)),
                  pl.BlockSpec((2, BN, 16), lambda i: (0, i, 0))]
                 + [_full_spec(a) for a in nw_ins],
        out_specs=[pl.BlockSpec((BN, 1), lambda i: (i, 0)),
                   pl.BlockSpec((BN, 2), lambda i: (i, 0))],
        out_shape=[jax.ShapeDtypeStruct((N, 1), f32),
                   jax.ShapeDtypeStruct((N, 2), f32)],
    )(x16, agg2, *nw_ins)

    edge_weights = w_full.reshape(E_PAD, 1)[:E]
    edge_mask = maskf.reshape(E_PAD)[:E].astype(bool)
    hit_mask = jnp.ones((N,), bool)
    return (edge_weights, H, beta, hit_mask, edge_mask)
